# baseline (speedup 1.0000x reference)
import functools

import jax
import jax.numpy as jnp
from jax import lax
from jax.experimental import pallas as pl
from jax.experimental.pallas import tpu as pltpu
from jax.experimental.pallas import tpu_sc as plsc

NC, NS = 2, 16
NW = NC * NS
NBW = 8
CB = 4096 // NBW
NHW = NW // NBW
CT = 400


@jax.jit
def _emb_lookup(xt, tt):
    h_total, b_total = xt.shape
    d, v_total = tt.shape
    h_per_w = h_total // NHW
    n_outer = h_per_w // 2
    n_vchunks = v_total // CT
    nk_lo = n_vchunks // NW
    n_rem = n_vchunks - nk_lo * NW

    mesh = plsc.VectorSubcoreMesh(
        core_axis_name="c", subcore_axis_name="s", num_cores=NC, num_subcores=NS
    )

    @functools.partial(
        pl.kernel,
        mesh=mesh,
        compiler_params=pltpu.CompilerParams(
            use_tc_tiling_on_sc=False, needs_layout_passes=False
        ),
        out_type=(
            jax.ShapeDtypeStruct((h_total, d, b_total), jnp.float32),
            jax.ShapeDtypeStruct((v_total, d), jnp.float32),
        ),
        scratch_types=[
            pltpu.VMEM((2, d, CT + 1), jnp.float32),
            pltpu.VMEM((2, CT, d), jnp.float32),
            pltpu.VMEM((2, CB), jnp.int32),
            pltpu.VMEM((2, CB, d), jnp.float32),
            pltpu.VMEM((2, d, CB + 1), jnp.float32),
            pltpu.SemaphoreType.DMA,
            pltpu.SemaphoreType.DMA,
            pltpu.SemaphoreType.DMA,
            pltpu.SemaphoreType.DMA,
            pltpu.SemaphoreType.DMA,
            pltpu.SemaphoreType.DMA,
            pltpu.SemaphoreType.REGULAR,
        ],
    )
    def body(
        xt_hbm, tt_hbm, out_hbm, trm_hbm,
        tbuf, rbuf, idx_v, rows_v, tr_v,
        s_st, s_wr, sg0, sg1, so0, so1, s_sync,
    ):
        cid = lax.axis_index("c")
        wid = lax.axis_index("s") * NC + cid
        lane = lax.iota(jnp.int32, 16)
        zeros16 = jnp.zeros((16,), jnp.int32)

        nk = jnp.where(wid < n_rem, nk_lo + 1, nk_lo)

        def stage_t(k):
            v0 = (wid + NW * k) * CT
            p = lax.rem(k, 2)
            pltpu.async_copy(
                tt_hbm.at[:, pl.ds(v0, CT)], tbuf.at[p].at[:, pl.ds(0, CT)],
                s_st,
            )

        def wait_stage():
            pltpu.make_async_copy(
                tt_hbm.at[:, pl.ds(0, CT)], tbuf.at[0].at[:, pl.ds(0, CT)],
                s_st,
            ).wait()

        def wait_write():
            pltpu.make_async_copy(
                rbuf.at[0], trm_hbm.at[pl.ds(0, CT)], s_wr
            ).wait()

        stage_t(0)

        def p1_body(k, carry):
            p = lax.rem(k, 2)
            wait_stage()
            lax.cond(k + 1 < nk, lambda: stage_t(k + 1), lambda: None)
            lax.cond(k >= 2, wait_write, lambda: None)

            @plsc.parallel_loop(0, CT, unroll=8)
            def trj(j):
                jv = zeros16 + j
                vlo = plsc.load_gather(tbuf.at[p], [lane, jv])
                vhi = plsc.load_gather(tbuf.at[p], [lane + 16, jv])
                rbuf[p, j, pl.ds(0, 16)] = vlo
                rbuf[p, j, pl.ds(16, 16)] = vhi

            v0 = (wid + NW * k) * CT
            pltpu.async_copy(rbuf.at[p], trm_hbm.at[pl.ds(v0, CT)], s_wr)
            return carry

        lax.fori_loop(0, nk, p1_body, 0)
        wait_write()
        wait_write()

        plsc.subcore_barrier()
        pltpu.semaphore_signal(s_sync, 1, core_index=1 - cid)
        pltpu.semaphore_wait(s_sync, 1)

        wb = lax.rem(wid, NBW)
        wh = lax.div(wid, NBW)
        b0 = wb * CB
        h_base = wh * h_per_w
        sg = [sg0, sg1]
        so = [so0, so1]

        def stage_idx(g, b):
            pltpu.sync_copy(xt_hbm.at[h_base + g, pl.ds(b0, CB)], idx_v.at[b])

        def fire_gathers(b):
            pltpu.async_copy(trm_hbm.at[idx_v.at[b]], rows_v.at[b], sg[b])

        def drain_gathers(b):
            pltpu.make_async_copy(
                trm_hbm.at[idx_v.at[b]], rows_v.at[b], sg[b]
            ).wait()

        def transpose_chunk(b):
            @plsc.parallel_loop(0, CB, unroll=8)
            def tr(j):
                jv = zeros16 + j
                lo = rows_v[b, j, pl.ds(0, 16)]
                hi = rows_v[b, j, pl.ds(16, 16)]
                plsc.store_scatter(tr_v.at[b], [lane, jv], lo)
                plsc.store_scatter(tr_v.at[b], [lane + 16, jv], hi)

        def fire_out(g, b):
            for dd in range(d):
                pltpu.async_copy(
                    tr_v.at[b].at[dd, pl.ds(0, CB)],
                    out_hbm.at[h_base + g, dd, pl.ds(b0, CB)],
                    so[b],
                )

        def wait_out(b):
            for dd in range(d):
                pltpu.make_async_copy(
                    tr_v.at[b].at[dd, pl.ds(0, CB)],
                    out_hbm.at[h_base, dd, pl.ds(b0, CB)],
                    so[b],
                ).wait()

        stage_idx(0, 0)
        fire_gathers(0)
        stage_idx(1, 1)

        def outer(t, carry):
            g0 = 2 * t
            drain_gathers(0)
            transpose_chunk(0)
            fire_out(g0, 0)
            lax.cond(t >= 1, lambda: wait_out(1), lambda: None)
            fire_gathers(1)
            lax.cond(
                t + 1 < n_outer, lambda: stage_idx(g0 + 2, 0), lambda: None
            )
            drain_gathers(1)
            transpose_chunk(1)
            fire_out(g0 + 1, 1)
            wait_out(0)
            lax.cond(t + 1 < n_outer, lambda: fire_gathers(0), lambda: None)
            lax.cond(
                t + 1 < n_outer, lambda: stage_idx(g0 + 3, 1), lambda: None
            )
            return carry

        lax.fori_loop(0, n_outer, outer, 0)
        wait_out(1)

    out, _ = body(xt, tt)
    return out


def kernel(x, table):
    xt = x.T
    tt = table.T
    outt = _emb_lookup(xt, tt)
    return jnp.transpose(outt, (2, 0, 1))


# speedup vs baseline: 4.1333x; 4.1333x over previous
import functools

import jax
import jax.numpy as jnp
from jax import lax
from jax.experimental import pallas as pl
from jax.experimental.pallas import tpu as pltpu
from jax.experimental.pallas import tpu_sc as plsc

NC, NS = 2, 16
NW = NC * NS
NBW = 8
CB = 4096 // NBW
NHW = NW // NBW


@jax.jit
def _emb_lookup(xt, table):
    h_total, b_total = xt.shape
    d = table.shape[1]
    h_per_w = h_total // NHW
    n_outer = h_per_w // 2

    mesh = plsc.VectorSubcoreMesh(
        core_axis_name="c", subcore_axis_name="s", num_cores=NC, num_subcores=NS
    )

    @functools.partial(
        pl.kernel,
        mesh=mesh,
        compiler_params=pltpu.CompilerParams(use_tc_tiling_on_sc=False, needs_layout_passes=False),
        out_type=jax.ShapeDtypeStruct((h_total, d, b_total), jnp.float32),
        scratch_types=[
            pltpu.VMEM((2, CB), jnp.int32),
            pltpu.VMEM((2, CB, d), jnp.float32),
            pltpu.VMEM((2, d, CB + 1), jnp.float32),
            pltpu.SemaphoreType.DMA,
            pltpu.SemaphoreType.DMA,
            pltpu.SemaphoreType.DMA,
            pltpu.SemaphoreType.DMA,
        ],
    )
    def body(
        xt_hbm, table_hbm, out_hbm, idx_v, rows_v, tr_v, sg0, sg1, so0, so1
    ):
        wid = lax.axis_index("s") * NC + lax.axis_index("c")
        wb = lax.rem(wid, NBW)
        wh = lax.div(wid, NBW)
        b0 = wb * CB
        h_base = wh * h_per_w
        sg = [sg0, sg1]
        so = [so0, so1]

        def stage_idx(g, b):
            pltpu.sync_copy(xt_hbm.at[h_base + g, pl.ds(b0, CB)], idx_v.at[b])

        def fire_gathers(b):
            pltpu.async_copy(table_hbm.at[idx_v.at[b]], rows_v.at[b], sg[b])

        def drain_gathers(b):
            pltpu.make_async_copy(
                table_hbm.at[idx_v.at[b]], rows_v.at[b], sg[b]
            ).wait()

        lane = lax.iota(jnp.int32, 16)
        zeros16 = jnp.zeros((16,), jnp.int32)

        def transpose_chunk(b):
            @plsc.parallel_loop(0, CB, unroll=8)
            def tr(j):
                jv = zeros16 + j
                lo = rows_v[b, j, pl.ds(0, 16)]
                hi = rows_v[b, j, pl.ds(16, 16)]
                plsc.store_scatter(tr_v.at[b], [lane, jv], lo)
                plsc.store_scatter(tr_v.at[b], [lane + 16, jv], hi)

        def fire_out(g, b):
            for dd in range(d):
                pltpu.async_copy(
                    tr_v.at[b].at[dd, pl.ds(0, CB)],
                    out_hbm.at[h_base + g, dd, pl.ds(b0, CB)],
                    so[b],
                )

        def wait_out(b):
            for dd in range(d):
                pltpu.make_async_copy(
                    tr_v.at[b].at[dd, pl.ds(0, CB)],
                    out_hbm.at[h_base, dd, pl.ds(b0, CB)],
                    so[b],
                ).wait()

        stage_idx(0, 0)
        fire_gathers(0)
        stage_idx(1, 1)

        def outer(t, carry):
            g0 = 2 * t
            drain_gathers(0)
            fire_gathers(1)
            lax.cond(
                t + 1 < n_outer, lambda: stage_idx(g0 + 2, 0), lambda: None
            )
            transpose_chunk(0)
            fire_out(g0, 0)
            lax.cond(t >= 1, lambda: wait_out(1), lambda: None)
            drain_gathers(1)
            lax.cond(t + 1 < n_outer, lambda: fire_gathers(0), lambda: None)
            lax.cond(
                t + 1 < n_outer, lambda: stage_idx(g0 + 3, 1), lambda: None
            )
            transpose_chunk(1)
            fire_out(g0 + 1, 1)
            wait_out(0)
            return carry

        lax.fori_loop(0, n_outer, outer, 0)
        wait_out(1)

    return body(xt, table)


def kernel(x, table):
    xt = x.T
    outt = _emb_lookup(xt, table)
    return jnp.transpose(outt, (2, 0, 1))
